# baseline (speedup 1.0000x reference)
import jax
import jax.numpy as jnp
from jax.experimental import pallas as pl

LATENT_DIM = 64
NUM_EMBED = 1024
RVQ_LAYERS = 4
BETA = 0.25

TILE = 256


def _conv(x, w, b, stride):
    y = jax.lax.conv_general_dilated(
        x, w, window_strides=(stride, stride), padding='SAME',
        dimension_numbers=('NHWC', 'HWIO', 'NHWC'))
    return y + b


def _conv_transpose(x, w, b, stride):
    y = jax.lax.conv_transpose(
        x, w, strides=(stride, stride), padding='SAME',
        dimension_numbers=('NHWC', 'HWIO', 'NHWC'))
    return y + b


def _rvq_kernel(z_ref, emb_ref, zq_ref, loss_ref):
    i = pl.program_id(0)
    z = z_ref[...]
    q = jnp.zeros_like(z)
    r = z
    for layer in range(RVQ_LAYERS):
        emb = emb_ref[layer]
        sim = jnp.dot(r, emb, preferred_element_type=jnp.float32)
        rn = jnp.sum(r * r, axis=1, keepdims=True)
        en = jnp.sum(emb * emb, axis=0, keepdims=True)
        dist = rn + en - 2.0 * sim
        idx = jnp.argmin(dist, axis=1)
        onehot = (jax.lax.broadcasted_iota(jnp.int32, (TILE, NUM_EMBED), 1)
                  == idx[:, None]).astype(jnp.float32)
        upd = jax.lax.dot_general(
            onehot, emb, (((1,), (1,)), ((), ())),
            preferred_element_type=jnp.float32,
            precision=jax.lax.Precision.HIGHEST)
        q = q + upd
        r = r - upd
    zq_ref[...] = q
    partial = jnp.sum(r * r).reshape(1, 1)

    @pl.when(i == 0)
    def _init():
        loss_ref[...] = partial

    @pl.when(i != 0)
    def _acc():
        loss_ref[...] += partial


def _fused_rvq(z_flat, embeddings):
    n = z_flat.shape[0]
    grid = n // TILE
    emb_l = jnp.transpose(embeddings, (2, 0, 1))
    zq, loss = pl.pallas_call(
        _rvq_kernel,
        grid=(grid,),
        in_specs=[
            pl.BlockSpec((TILE, LATENT_DIM), lambda i: (i, 0)),
            pl.BlockSpec((RVQ_LAYERS, LATENT_DIM, NUM_EMBED),
                         lambda i: (0, 0, 0)),
        ],
        out_specs=[
            pl.BlockSpec((TILE, LATENT_DIM), lambda i: (i, 0)),
            pl.BlockSpec((1, 1), lambda i: (0, 0)),
        ],
        out_shape=[
            jax.ShapeDtypeStruct((n, LATENT_DIM), jnp.float32),
            jax.ShapeDtypeStruct((1, 1), jnp.float32),
        ],
    )(z_flat, emb_l)
    vq_loss = BETA * loss[0, 0] / (n * LATENT_DIM)
    return zq, vq_loss


def kernel(x, enc_w1, enc_b1, enc_w2, enc_b2, enc_w3, enc_b3,
           dec_w1, dec_b1, dec_w2, dec_b2, dec_w3, dec_b3, embeddings):
    h = jax.nn.relu(_conv(x, enc_w1, enc_b1, 2))
    h = jax.nn.relu(_conv(h, enc_w2, enc_b2, 2))
    z = _conv(h, enc_w3, enc_b3, 1)
    b, hh, ww, _ = z.shape
    zq_flat, vq_loss = _fused_rvq(z.reshape(-1, LATENT_DIM), embeddings)
    zq = zq_flat.reshape(b, hh, ww, LATENT_DIM)
    d = jax.nn.relu(_conv_transpose(zq, dec_w1, dec_b1, 2))
    d = jax.nn.relu(_conv_transpose(d, dec_w2, dec_b2, 2))
    recon = _conv_transpose(d, dec_w3, dec_b3, 1)
    return recon, vq_loss


# speedup vs baseline: 1.0298x; 1.0298x over previous
import jax
import jax.numpy as jnp
from jax.experimental import pallas as pl

LATENT_DIM = 64
NUM_EMBED = 1024
RVQ_LAYERS = 4
BETA = 0.25

TILE = 512


def _conv(x, w, b, stride):
    y = jax.lax.conv_general_dilated(
        x, w, window_strides=(stride, stride), padding='SAME',
        dimension_numbers=('NHWC', 'HWIO', 'NHWC'))
    return y + b


def _conv_transpose(x, w, b, stride):
    y = jax.lax.conv_transpose(
        x, w, strides=(stride, stride), padding='SAME',
        dimension_numbers=('NHWC', 'HWIO', 'NHWC'))
    return y + b


def _rvq_kernel(z_ref, emb_ref, embt_ref, zq_ref, loss_ref):
    i = pl.program_id(0)
    z = z_ref[...]
    q = jnp.zeros_like(z)
    r = z
    for layer in range(RVQ_LAYERS):
        emb = emb_ref[layer]
        sim = jnp.dot(r, emb, preferred_element_type=jnp.float32)
        rn = jnp.sum(r * r, axis=1, keepdims=True)
        en = jnp.sum(emb * emb, axis=0, keepdims=True)
        dist = rn + en - 2.0 * sim
        m = jnp.min(dist, axis=1, keepdims=True)
        iota = jax.lax.broadcasted_iota(jnp.int32, (TILE, NUM_EMBED), 1)
        cand = jnp.where(dist == m, iota, NUM_EMBED)
        idx = jnp.min(cand, axis=1, keepdims=True)
        onehot = (iota == idx).astype(jnp.float32)
        upd = jnp.dot(onehot, embt_ref[layer],
                      preferred_element_type=jnp.float32,
                      precision=jax.lax.Precision.HIGHEST)
        q = q + upd
        r = r - upd
    zq_ref[...] = q
    partial = jnp.sum(r * r).reshape(1, 1)

    @pl.when(i == 0)
    def _init():
        loss_ref[...] = partial

    @pl.when(i != 0)
    def _acc():
        loss_ref[...] += partial


def _fused_rvq(z_flat, embeddings):
    n = z_flat.shape[0]
    grid = n // TILE
    emb_l = jnp.transpose(embeddings, (2, 0, 1))
    embt_l = jnp.transpose(embeddings, (2, 1, 0))
    zq, loss = pl.pallas_call(
        _rvq_kernel,
        grid=(grid,),
        in_specs=[
            pl.BlockSpec((TILE, LATENT_DIM), lambda i: (i, 0)),
            pl.BlockSpec((RVQ_LAYERS, LATENT_DIM, NUM_EMBED),
                         lambda i: (0, 0, 0)),
            pl.BlockSpec((RVQ_LAYERS, NUM_EMBED, LATENT_DIM),
                         lambda i: (0, 0, 0)),
        ],
        out_specs=[
            pl.BlockSpec((TILE, LATENT_DIM), lambda i: (i, 0)),
            pl.BlockSpec((1, 1), lambda i: (0, 0)),
        ],
        out_shape=[
            jax.ShapeDtypeStruct((n, LATENT_DIM), jnp.float32),
            jax.ShapeDtypeStruct((1, 1), jnp.float32),
        ],
    )(z_flat, emb_l, embt_l)
    vq_loss = BETA * loss[0, 0] / (n * LATENT_DIM)
    return zq, vq_loss


def kernel(x, enc_w1, enc_b1, enc_w2, enc_b2, enc_w3, enc_b3,
           dec_w1, dec_b1, dec_w2, dec_b2, dec_w3, dec_b3, embeddings):
    h = jax.nn.relu(_conv(x, enc_w1, enc_b1, 2))
    h = jax.nn.relu(_conv(h, enc_w2, enc_b2, 2))
    z = _conv(h, enc_w3, enc_b3, 1)
    b, hh, ww, _ = z.shape
    zq_flat, vq_loss = _fused_rvq(z.reshape(-1, LATENT_DIM), embeddings)
    zq = zq_flat.reshape(b, hh, ww, LATENT_DIM)
    d = jax.nn.relu(_conv_transpose(zq, dec_w1, dec_b1, 2))
    d = jax.nn.relu(_conv_transpose(d, dec_w2, dec_b2, 2))
    recon = _conv_transpose(d, dec_w3, dec_b3, 1)
    return recon, vq_loss


# speedup vs baseline: 1.6598x; 1.6117x over previous
import jax
import jax.numpy as jnp
from jax.experimental import pallas as pl

LATENT_DIM = 64
NUM_EMBED = 1024
RVQ_LAYERS = 4
BETA = 0.25

TILE = 512


def _conv(x, w, b, stride):
    y = jax.lax.conv_general_dilated(
        x, w, window_strides=(stride, stride), padding='SAME',
        dimension_numbers=('NHWC', 'HWIO', 'NHWC'))
    return y + b


def _conv_transpose(x, w, b, stride):
    y = jax.lax.conv_transpose(
        x, w, strides=(stride, stride), padding='SAME',
        dimension_numbers=('NHWC', 'HWIO', 'NHWC'))
    return y + b


def _rvq_kernel(z_ref, emb_ref, embt_hi_ref, embt_lo_ref, zq_ref, loss_ref):
    i = pl.program_id(0)
    z = z_ref[...]
    q = jnp.zeros_like(z)
    r = z
    for layer in range(RVQ_LAYERS):
        emb = emb_ref[layer]
        sim = jnp.dot(r, emb, preferred_element_type=jnp.float32)
        rn = jnp.sum(r * r, axis=1, keepdims=True)
        en = jnp.sum(emb * emb, axis=0, keepdims=True)
        dist = rn + en - 2.0 * sim
        m = jnp.min(dist, axis=1, keepdims=True)
        iota = jax.lax.broadcasted_iota(jnp.int32, (TILE, NUM_EMBED), 1)
        cand = jnp.where(dist == m, iota, NUM_EMBED)
        idx = jnp.min(cand, axis=1, keepdims=True)
        onehot = (iota == idx).astype(jnp.float32)
        upd = (jnp.dot(onehot, embt_hi_ref[layer],
                       preferred_element_type=jnp.float32)
               + jnp.dot(onehot, embt_lo_ref[layer],
                         preferred_element_type=jnp.float32))
        q = q + upd
        r = r - upd
    zq_ref[...] = q
    partial = jnp.sum(r * r).reshape(1, 1)

    @pl.when(i == 0)
    def _init():
        loss_ref[...] = partial

    @pl.when(i != 0)
    def _acc():
        loss_ref[...] += partial


def _fused_rvq(z_flat, embeddings):
    n = z_flat.shape[0]
    grid = n // TILE
    emb_l = jnp.transpose(embeddings, (2, 0, 1))
    embt_l = jnp.transpose(embeddings, (2, 1, 0))
    embt_hi = embt_l.astype(jnp.bfloat16).astype(jnp.float32)
    embt_lo = embt_l - embt_hi
    zq, loss = pl.pallas_call(
        _rvq_kernel,
        grid=(grid,),
        in_specs=[
            pl.BlockSpec((TILE, LATENT_DIM), lambda i: (i, 0)),
            pl.BlockSpec((RVQ_LAYERS, LATENT_DIM, NUM_EMBED),
                         lambda i: (0, 0, 0)),
            pl.BlockSpec((RVQ_LAYERS, NUM_EMBED, LATENT_DIM),
                         lambda i: (0, 0, 0)),
            pl.BlockSpec((RVQ_LAYERS, NUM_EMBED, LATENT_DIM),
                         lambda i: (0, 0, 0)),
        ],
        out_specs=[
            pl.BlockSpec((TILE, LATENT_DIM), lambda i: (i, 0)),
            pl.BlockSpec((1, 1), lambda i: (0, 0)),
        ],
        out_shape=[
            jax.ShapeDtypeStruct((n, LATENT_DIM), jnp.float32),
            jax.ShapeDtypeStruct((1, 1), jnp.float32),
        ],
    )(z_flat, emb_l, embt_hi, embt_lo)
    vq_loss = BETA * loss[0, 0] / (n * LATENT_DIM)
    return zq, vq_loss


def kernel(x, enc_w1, enc_b1, enc_w2, enc_b2, enc_w3, enc_b3,
           dec_w1, dec_b1, dec_w2, dec_b2, dec_w3, dec_b3, embeddings):
    h = jax.nn.relu(_conv(x, enc_w1, enc_b1, 2))
    h = jax.nn.relu(_conv(h, enc_w2, enc_b2, 2))
    z = _conv(h, enc_w3, enc_b3, 1)
    b, hh, ww, _ = z.shape
    zq_flat, vq_loss = _fused_rvq(z.reshape(-1, LATENT_DIM), embeddings)
    zq = zq_flat.reshape(b, hh, ww, LATENT_DIM)
    d = jax.nn.relu(_conv_transpose(zq, dec_w1, dec_b1, 2))
    d = jax.nn.relu(_conv_transpose(d, dec_w2, dec_b2, 2))
    recon = _conv_transpose(d, dec_w3, dec_b3, 1)
    return recon, vq_loss
